# baseline (speedup 1.0000x reference)
import functools

import jax
import jax.numpy as jnp
from jax import lax
from jax.experimental import pallas as pl
from jax.experimental.pallas import tpu as pltpu, tpu_sc as plsc

_INFO = plsc.get_sparse_core_info()
_NC, _NS, _L = _INFO.num_cores, _INFO.num_subcores, _INFO.num_lanes
_NW = _NC * _NS

_N = 4096 * 2048
_PER_W = _N // _NW
_CHUNK = 16384
_NCHUNK = _PER_W // _CHUNK
_UNROLL = 4


def _sc_body(x_hbm, t_hbm, w_hbm, out_hbm, xb, tb, wv, accv, sems, wsem):
    wid = lax.axis_index("s") * _NC + lax.axis_index("c")
    base = wid * _PER_W

    pltpu.async_copy(w_hbm, wv, wsem).wait()

    def start(g, buf):
        off = base + g * _CHUNK
        cx = pltpu.async_copy(x_hbm.at[pl.ds(off, _CHUNK)], xb.at[buf],
                              sems.at[buf])
        ct = pltpu.async_copy(t_hbm.at[pl.ds(off, _CHUNK)], tb.at[buf],
                              sems.at[buf])
        return cx, ct

    pend = start(0, 0)
    acc = jnp.zeros((_L,), jnp.float32)

    for g in range(_NCHUNK):
        buf = g % 2
        if g + 1 < _NCHUNK:
            nxt = start(g + 1, (g + 1) % 2)
        pend[0].wait()
        pend[1].wait()

        def body(i, acc, buf=buf):
            b = i * (_L * _UNROLL)
            for u in range(_UNROLL):
                x = xb[buf, pl.ds(b + u * _L, _L)]
                t = tb[buf, pl.ds(b + u * _L, _L)]
                w = plsc.load_gather(wv, [t])
                d = x - t.astype(jnp.float32)
                acc = acc + w * (d * d)
            return acc

        acc = lax.fori_loop(0, _CHUNK // (_L * _UNROLL), body, acc)
        if g + 1 < _NCHUNK:
            pend = nxt

    accv[...] = acc
    pltpu.sync_copy(accv, out_hbm.at[wid])


@jax.jit
def kernel(inputs, targets, weight):
    x = inputs.reshape(_N)
    t = targets.reshape(_N)
    partials = pl.kernel(
        _sc_body,
        out_type=jax.ShapeDtypeStruct((_NW, _L), jnp.float32),
        mesh=plsc.VectorSubcoreMesh(core_axis_name="c", subcore_axis_name="s"),
        compiler_params=pltpu.CompilerParams(needs_layout_passes=False),
        scratch_types=[
            pltpu.VMEM((2, _CHUNK), jnp.float32),
            pltpu.VMEM((2, _CHUNK), jnp.int32),
            pltpu.VMEM((_L,), jnp.float32),
            pltpu.VMEM((_L,), jnp.float32),
            pltpu.SemaphoreType.DMA((2,)),
            pltpu.SemaphoreType.DMA,
        ],
    )(x, t, weight)
    return jnp.sum(partials) * (1.0 / _N)


# speedup vs baseline: 2.1951x; 2.1951x over previous
import jax
import jax.numpy as jnp
from jax import lax
from jax.experimental import pallas as pl
from jax.experimental.pallas import tpu as pltpu, tpu_sc as plsc

_INFO = plsc.get_sparse_core_info()
_NC, _NS, _L = _INFO.num_cores, _INFO.num_subcores, _INFO.num_lanes
_NW = _NC * _NS

_ROWS, _COLS = 4096, 2048
_N = _ROWS * _COLS
_ROWS_W = _ROWS // _NW
_CROWS = 8
_NCHUNK = _ROWS_W // _CROWS


def _sc_body(x_hbm, t_hbm, w_hbm, out_hbm, xb, tb, wv, accv, sems, wsem):
    wid = lax.axis_index("s") * _NC + lax.axis_index("c")
    row0 = wid * _ROWS_W

    pltpu.async_copy(w_hbm, wv, wsem).wait()

    def start(g, buf):
        r = row0 + g * _CROWS
        cx = pltpu.async_copy(x_hbm.at[pl.ds(r, _CROWS)], xb.at[buf],
                              sems.at[buf])
        ct = pltpu.async_copy(t_hbm.at[pl.ds(r, _CROWS)], tb.at[buf],
                              sems.at[buf])
        return cx, ct

    pend = start(0, 0)
    acc = jnp.zeros((_L,), jnp.float32)

    for g in range(_NCHUNK):
        buf = g % 2
        if g + 1 < _NCHUNK:
            nxt = start(g + 1, (g + 1) % 2)
        pend[0].wait()
        pend[1].wait()

        def body(i, acc, buf=buf):
            c = i * _L
            for r in range(_CROWS):
                x = xb[buf, r, pl.ds(c, _L)]
                t = tb[buf, r, pl.ds(c, _L)]
                w = plsc.load_gather(wv, [t])
                d = x - t.astype(jnp.float32)
                acc = acc + w * (d * d)
            return acc

        acc = lax.fori_loop(0, _COLS // _L, body, acc)
        if g + 1 < _NCHUNK:
            pend = nxt

    accv[...] = acc
    pltpu.sync_copy(accv, out_hbm.at[wid])


@jax.jit
def kernel(inputs, targets, weight):
    partials = pl.kernel(
        _sc_body,
        out_type=jax.ShapeDtypeStruct((_NW, _L), jnp.float32),
        mesh=plsc.VectorSubcoreMesh(core_axis_name="c", subcore_axis_name="s"),
        compiler_params=pltpu.CompilerParams(needs_layout_passes=False),
        scratch_types=[
            pltpu.VMEM((2, _CROWS, _COLS), jnp.float32),
            pltpu.VMEM((2, _CROWS, _COLS), jnp.int32),
            pltpu.VMEM((_L,), jnp.float32),
            pltpu.VMEM((_L,), jnp.float32),
            pltpu.SemaphoreType.DMA((2,)),
            pltpu.SemaphoreType.DMA,
        ],
    )(inputs, targets, weight)
    return jnp.sum(partials) * (1.0 / _N)


# speedup vs baseline: 2.1991x; 1.0018x over previous
import jax
import jax.numpy as jnp
from jax import lax
from jax.experimental import pallas as pl
from jax.experimental.pallas import tpu as pltpu, tpu_sc as plsc

_INFO = plsc.get_sparse_core_info()
_NC, _NS, _L = _INFO.num_cores, _INFO.num_subcores, _INFO.num_lanes
_NW = _NC * _NS

_ROWS, _COLS = 4096, 2048
_N = _ROWS * _COLS
_ROWS_W = _ROWS // _NW
_CROWS = 8
_NCHUNK = _ROWS_W // _CROWS


def _sc_body(x_hbm, t_hbm, w_hbm, out_hbm, xb, tb, wv, accv, sems, wsem):
    wid = lax.axis_index("s") * _NC + lax.axis_index("c")
    row0 = wid * _ROWS_W

    pltpu.async_copy(w_hbm, wv, wsem).wait()

    def start(g, buf):
        r = row0 + g * _CROWS
        cx = pltpu.async_copy(x_hbm.at[pl.ds(r, _CROWS)], xb.at[buf],
                              sems.at[buf])
        ct = pltpu.async_copy(t_hbm.at[pl.ds(r, _CROWS)], tb.at[buf],
                              sems.at[buf])
        return cx, ct

    pend = start(0, 0)
    accs = (jnp.zeros((_L,), jnp.float32),) * _CROWS

    for g in range(_NCHUNK):
        buf = g % 2
        if g + 1 < _NCHUNK:
            nxt = start(g + 1, (g + 1) % 2)
        pend[0].wait()
        pend[1].wait()

        def body(i, accs, buf=buf):
            c = i * _L
            new = []
            for r in range(_CROWS):
                x = xb[buf, r, pl.ds(c, _L)]
                t = tb[buf, r, pl.ds(c, _L)]
                w = plsc.load_gather(wv, [t])
                d = x - t.astype(jnp.float32)
                new.append(accs[r] + w * (d * d))
            return tuple(new)

        accs = lax.fori_loop(0, _COLS // _L, body, accs)
        if g + 1 < _NCHUNK:
            pend = nxt

    acc = accs[0]
    for r in range(1, _CROWS):
        acc = acc + accs[r]
    accv[...] = acc
    pltpu.sync_copy(accv, out_hbm.at[wid])


@jax.jit
def kernel(inputs, targets, weight):
    partials = pl.kernel(
        _sc_body,
        out_type=jax.ShapeDtypeStruct((_NW, _L), jnp.float32),
        mesh=plsc.VectorSubcoreMesh(core_axis_name="c", subcore_axis_name="s"),
        compiler_params=pltpu.CompilerParams(needs_layout_passes=False),
        scratch_types=[
            pltpu.VMEM((2, _CROWS, _COLS), jnp.float32),
            pltpu.VMEM((2, _CROWS, _COLS), jnp.int32),
            pltpu.VMEM((_L,), jnp.float32),
            pltpu.VMEM((_L,), jnp.float32),
            pltpu.SemaphoreType.DMA((2,)),
            pltpu.SemaphoreType.DMA,
        ],
    )(inputs, targets, weight)
    return jnp.sum(partials) * (1.0 / _N)


# speedup vs baseline: 2.5818x; 1.1741x over previous
import jax
import jax.numpy as jnp
from jax import lax
from jax.experimental import pallas as pl
from jax.experimental.pallas import tpu as pltpu, tpu_sc as plsc

_INFO = plsc.get_sparse_core_info()
_NC, _NS, _L = _INFO.num_cores, _INFO.num_subcores, _INFO.num_lanes
_NW = _NC * _NS

_ROWS, _COLS = 4096, 2048
_N = _ROWS * _COLS

_SC_ROWS = 2048
_ROWS_W = _SC_ROWS // _NW
_CROWS = 8
_NCHUNK = _ROWS_W // _CROWS

_TC_ROWS = _ROWS - _SC_ROWS
_TC_BR = 256
_TC_NB = _TC_ROWS // _TC_BR


def _sc_body(x_hbm, t_hbm, w_hbm, out_hbm, xb, tb, wv, accv, sems, wsem):
    wid = lax.axis_index("s") * _NC + lax.axis_index("c")
    row0 = wid * _ROWS_W

    pltpu.async_copy(w_hbm, wv, wsem).wait()

    def start(g, buf):
        r = row0 + g * _CROWS
        cx = pltpu.async_copy(x_hbm.at[pl.ds(r, _CROWS)], xb.at[buf],
                              sems.at[buf])
        ct = pltpu.async_copy(t_hbm.at[pl.ds(r, _CROWS)], tb.at[buf],
                              sems.at[buf])
        return cx, ct

    pend = start(0, 0)
    accs = (jnp.zeros((_L,), jnp.float32),) * _CROWS

    for g in range(_NCHUNK):
        buf = g % 2
        if g + 1 < _NCHUNK:
            nxt = start(g + 1, (g + 1) % 2)
        pend[0].wait()
        pend[1].wait()

        def body(i, accs, buf=buf):
            c = i * _L
            new = []
            for r in range(_CROWS):
                x = xb[buf, r, pl.ds(c, _L)]
                t = tb[buf, r, pl.ds(c, _L)]
                w = plsc.load_gather(wv, [t])
                d = x - t.astype(jnp.float32)
                new.append(accs[r] + w * (d * d))
            return tuple(new)

        accs = lax.fori_loop(0, _COLS // _L, body, accs)
        if g + 1 < _NCHUNK:
            pend = nxt

    acc = accs[0]
    for r in range(1, _CROWS):
        acc = acc + accs[r]
    accv[...] = acc
    pltpu.sync_copy(accv, out_hbm.at[wid])


def _sc_partials(inputs, targets, weight):
    return pl.kernel(
        _sc_body,
        out_type=jax.ShapeDtypeStruct((_NW, _L), jnp.float32),
        mesh=plsc.VectorSubcoreMesh(core_axis_name="c", subcore_axis_name="s"),
        compiler_params=pltpu.CompilerParams(needs_layout_passes=False),
        scratch_types=[
            pltpu.VMEM((2, _CROWS, _COLS), jnp.float32),
            pltpu.VMEM((2, _CROWS, _COLS), jnp.int32),
            pltpu.VMEM((_L,), jnp.float32),
            pltpu.VMEM((_L,), jnp.float32),
            pltpu.SemaphoreType.DMA((2,)),
            pltpu.SemaphoreType.DMA,
        ],
    )(inputs, targets, weight)


def _tc_body(w_smem, x_ref, t_ref, out_smem):
    j = pl.program_id(0)
    x = x_ref[...]
    t = t_ref[...]
    d = x - t.astype(jnp.float32)
    d2 = d * d
    bits = [(t >> k) & 1 == 1 for k in range(4)]
    cur = [jnp.where(bits[0], w_smem[2 * i + 1], w_smem[2 * i])
           for i in range(8)]
    for k in range(1, 4):
        cur = [jnp.where(bits[k], cur[2 * i + 1], cur[2 * i])
               for i in range(len(cur) // 2)]
    s = jnp.sum(cur[0] * d2)

    @pl.when(j == 0)
    def _():
        out_smem[0] = 0.0

    out_smem[0] += s


def _tc_partial(inputs, targets, weight):
    return pl.pallas_call(
        _tc_body,
        grid=(_TC_NB,),
        in_specs=[
            pl.BlockSpec(memory_space=pltpu.SMEM),
            pl.BlockSpec((_TC_BR, _COLS),
                         lambda j: (_SC_ROWS // _TC_BR + j, 0)),
            pl.BlockSpec((_TC_BR, _COLS),
                         lambda j: (_SC_ROWS // _TC_BR + j, 0)),
        ],
        out_specs=pl.BlockSpec(memory_space=pltpu.SMEM),
        out_shape=jax.ShapeDtypeStruct((1,), jnp.float32),
        compiler_params=pltpu.CompilerParams(
            dimension_semantics=("arbitrary",)),
    )(weight, inputs, targets)


@jax.jit
def kernel(inputs, targets, weight):
    sc = _sc_partials(inputs, targets, weight)
    tc = _tc_partial(inputs, targets, weight)
    return (jnp.sum(sc) + tc[0]) * (1.0 / _N)
